# baseline (speedup 1.0000x reference)
import functools

import jax
import jax.numpy as jnp
from jax import lax
from jax.experimental import pallas as pl
from jax.experimental.pallas import tpu as pltpu
from jax.experimental.pallas import tpu_sc as plsc

N = 16384
D = 128
DH = 64
NW = 32
ROWS_PER_W = N // NW
CHUNK = 256


def _sc_gather_body(idx, table, out, idx_v, rows, sem):
    wid = lax.axis_index("s") * 2 + lax.axis_index("c")
    base = wid * ROWS_PER_W
    pltpu.sync_copy(idx.at[pl.ds(base, ROWS_PER_W)], idx_v)

    @pl.loop(0, ROWS_PER_W, step=16)
    def _(r):
        v = idx_v[pl.ds(r, 16)]
        for k in range(16):
            pltpu.async_copy(
                table.at[pl.ds(v[k], 1)], rows.at[pl.ds(r + k, 1)], sem)

    pltpu.make_async_copy(table.at[pl.ds(0, ROWS_PER_W)], rows, sem).wait()
    pltpu.sync_copy(rows, out.at[pl.ds(base, ROWS_PER_W)])


def _sc_gather(idx1d, table):
    mesh = plsc.VectorSubcoreMesh(core_axis_name="c", subcore_axis_name="s")
    fn = pl.kernel(
        _sc_gather_body,
        mesh=mesh,
        out_type=jax.ShapeDtypeStruct((N, DH), jnp.float32),
        scratch_types=[
            pltpu.VMEM((ROWS_PER_W,), jnp.int32),
            pltpu.VMEM((ROWS_PER_W, DH), jnp.float32),
            pltpu.SemaphoreType.DMA,
        ],
    )
    return fn(idx1d, table)


def _ln(x, g, b, eps=1e-5):
    m = jnp.mean(x, axis=-1, keepdims=True)
    v = jnp.mean((x - m) ** 2, axis=-1, keepdims=True)
    return (x - m) * lax.rsqrt(v + eps) * g + b


def _tc_body(nf_ref, act_ref, res_ref, w1_ref, b1_ref, nng_ref, nnb_ref,
             ln1g_ref, ln1b_ref, wpa_ref, wpb_ref, wpc_ref, bp_ref,
             ln2g_ref, ln2b_ref, out_ref):
    xn = _ln(nf_ref[...], nng_ref[...], nnb_ref[...])
    x = jnp.maximum(
        jnp.dot(xn, w1_ref[...], preferred_element_type=jnp.float32)
        + b1_ref[...], 0.0)
    num_emb = _ln(x, ln1g_ref[...], ln1b_ref[...])
    acc = jnp.dot(act_ref[...], wpa_ref[...], preferred_element_type=jnp.float32)
    acc = acc + jnp.dot(res_ref[...], wpb_ref[...], preferred_element_type=jnp.float32)
    acc = acc + jnp.dot(num_emb, wpc_ref[...], preferred_element_type=jnp.float32)
    o = jnp.maximum(acc + bp_ref[...], 0.0)
    out_ref[...] = _ln(o, ln2g_ref[...], ln2b_ref[...])


def _tc_fused(num_feats, act_emb, res_emb, nn_g, nn_b, W1, b1, ln1_g, ln1_b,
              Wp, bp, ln2_g, ln2_b, block_n=4096):
    grid = (N // block_n,)
    row_blk = lambda cols: pl.BlockSpec((block_n, cols), lambda i: (i, 0))
    full = lambda shape: pl.BlockSpec(shape, lambda i: (0, 0))
    return pl.pallas_call(
        _tc_body,
        grid=grid,
        in_specs=[
            row_blk(3),
            row_blk(DH),
            row_blk(DH),
            full((3, D)),
            full((1, D)),
            full((1, 3)),
            full((1, 3)),
            full((1, D)),
            full((1, D)),
            full((DH, D)),
            full((DH, D)),
            full((D, D)),
            full((1, D)),
            full((1, D)),
            full((1, D)),
        ],
        out_specs=row_blk(D),
        out_shape=jax.ShapeDtypeStruct((N, D), jnp.float32),
    )(num_feats, act_emb, res_emb, W1, b1.reshape(1, D), nn_g.reshape(1, 3),
      nn_b.reshape(1, 3), ln1_g.reshape(1, D), ln1_b.reshape(1, D),
      Wp[:DH], Wp[DH:D], Wp[D:], bp.reshape(1, D), ln2_g.reshape(1, D),
      ln2_b.reshape(1, D))


def kernel(activities, resources, num_feats, act_table, res_table, nn_g,
           nn_b, W1, b1, ln1_g, ln1_b, Wp, bp, ln2_g, ln2_b):
    act_emb = _sc_gather(activities.astype(jnp.int32), act_table)
    res_emb = _sc_gather(resources.astype(jnp.int32), res_table)
    return _tc_fused(num_feats, act_emb, res_emb, nn_g, nn_b, W1, b1,
                     ln1_g, ln1_b, Wp, bp, ln2_g, ln2_b)


# speedup vs baseline: 1.0582x; 1.0582x over previous
import functools

import jax
import jax.numpy as jnp
from jax import lax
from jax.experimental import pallas as pl
from jax.experimental.pallas import tpu as pltpu
from jax.experimental.pallas import tpu_sc as plsc

N = 16384
D = 128
DH = 64
NW = 32
ROWS_PER_W = N // NW
CHUNK = 256


def _sc_gather_body(idx, table, out, idx_v, rows, sem):
    wid = lax.axis_index("s") * 2 + lax.axis_index("c")
    base = wid * ROWS_PER_W
    pltpu.sync_copy(idx.at[pl.ds(base, ROWS_PER_W)], idx_v)

    @pl.loop(0, ROWS_PER_W, step=16)
    def _(r):
        v = idx_v[pl.ds(r, 16)]
        for k in range(16):
            pltpu.async_copy(
                table.at[pl.ds(v[k], 1)], rows.at[pl.ds(r + k, 1)], sem)

    pltpu.make_async_copy(table.at[pl.ds(0, ROWS_PER_W)], rows, sem).wait()
    pltpu.sync_copy(rows, out.at[pl.ds(base, ROWS_PER_W)])


def _sc_gather(idx1d, table):
    mesh = plsc.VectorSubcoreMesh(core_axis_name="c", subcore_axis_name="s")
    fn = pl.kernel(
        _sc_gather_body,
        mesh=mesh,
        out_type=jax.ShapeDtypeStruct((N, DH), jnp.float32),
        scratch_types=[
            pltpu.VMEM((ROWS_PER_W,), jnp.int32),
            pltpu.VMEM((ROWS_PER_W, DH), jnp.float32),
            pltpu.SemaphoreType.DMA,
        ],
    )
    return fn(idx1d, table)


def _ln(x, g, b, axis, eps=1e-5):
    m = jnp.mean(x, axis=axis, keepdims=True)
    v = jnp.mean((x - m) ** 2, axis=axis, keepdims=True)
    return (x - m) * lax.rsqrt(v + eps) * g + b


def _tc_body(nft_ref, act_ref, res_ref, w1_ref, b1_ref, nng_ref, nnb_ref,
             ln1g_ref, ln1b_ref, wpa_ref, wpb_ref, wpc_ref, bp_ref,
             ln2g_ref, ln2b_ref, out_ref):
    xnt = _ln(nft_ref[...], nng_ref[...], nnb_ref[...], axis=0)
    h = lax.dot_general(w1_ref[...], xnt, (((0,), (0,)), ((), ())),
                        preferred_element_type=jnp.float32)
    h = jnp.maximum(h + b1_ref[...], 0.0)
    num_embt = _ln(h, ln1g_ref[...], ln1b_ref[...], axis=0)
    acc = lax.dot_general(num_embt, wpc_ref[...], (((0,), (0,)), ((), ())),
                          preferred_element_type=jnp.float32)
    acc = acc + jnp.dot(act_ref[...], wpa_ref[...],
                        preferred_element_type=jnp.float32)
    acc = acc + jnp.dot(res_ref[...], wpb_ref[...],
                        preferred_element_type=jnp.float32)
    o = jnp.maximum(acc + bp_ref[...], 0.0)
    out_ref[...] = _ln(o, ln2g_ref[...], ln2b_ref[...], axis=-1)


def _tc_fused(num_feats_t, act_emb, res_emb, nn_g, nn_b, W1, b1, ln1_g,
              ln1_b, Wp, bp, ln2_g, ln2_b, block_n=4096):
    grid = (N // block_n,)
    row_blk = lambda cols: pl.BlockSpec((block_n, cols), lambda i: (i, 0))
    col_blk = lambda rows: pl.BlockSpec((rows, block_n), lambda i: (0, i))
    full = lambda shape: pl.BlockSpec(shape, lambda i: (0, 0))
    return pl.pallas_call(
        _tc_body,
        grid=grid,
        in_specs=[
            col_blk(3),
            row_blk(DH),
            row_blk(DH),
            full((3, D)),
            full((D, 1)),
            full((3, 1)),
            full((3, 1)),
            full((D, 1)),
            full((D, 1)),
            full((DH, D)),
            full((DH, D)),
            full((D, D)),
            full((1, D)),
            full((1, D)),
            full((1, D)),
        ],
        out_specs=row_blk(D),
        out_shape=jax.ShapeDtypeStruct((N, D), jnp.float32),
    )(num_feats_t, act_emb, res_emb, W1, b1.reshape(D, 1),
      nn_g.reshape(3, 1), nn_b.reshape(3, 1), ln1_g.reshape(D, 1),
      ln1_b.reshape(D, 1), Wp[:DH], Wp[DH:D], Wp[D:], bp.reshape(1, D),
      ln2_g.reshape(1, D), ln2_b.reshape(1, D))


def kernel(activities, resources, num_feats, act_table, res_table, nn_g,
           nn_b, W1, b1, ln1_g, ln1_b, Wp, bp, ln2_g, ln2_b):
    act_emb = _sc_gather(activities.astype(jnp.int32), act_table)
    res_emb = _sc_gather(resources.astype(jnp.int32), res_table)
    return _tc_fused(num_feats.T, act_emb, res_emb, nn_g, nn_b, W1, b1,
                     ln1_g, ln1_b, Wp, bp, ln2_g, ln2_b)


# speedup vs baseline: 1.0635x; 1.0050x over previous
import functools

import jax
import jax.numpy as jnp
from jax import lax
from jax.experimental import pallas as pl
from jax.experimental.pallas import tpu as pltpu
from jax.experimental.pallas import tpu_sc as plsc

N = 16384
D = 128
DH = 64
NW = 32
ROWS_PER_W = N // NW
CHUNK = 256


def _sc_gather_body(idx, table, out, idx_v, rows, sem):
    wid = lax.axis_index("s") * 2 + lax.axis_index("c")
    base = wid * ROWS_PER_W
    pltpu.sync_copy(idx.at[pl.ds(base, ROWS_PER_W)], idx_v)

    @pl.loop(0, ROWS_PER_W, step=32)
    def _(r):
        for g in range(2):
            v = idx_v[pl.ds(r + g * 16, 16)]
            for k in range(16):
                pltpu.async_copy(
                    table.at[pl.ds(v[k], 1)],
                    rows.at[pl.ds(r + g * 16 + k, 1)], sem)

    pltpu.make_async_copy(table.at[pl.ds(0, ROWS_PER_W)], rows, sem).wait()
    pltpu.sync_copy(rows, out.at[pl.ds(base, ROWS_PER_W)])


def _sc_gather(idx1d, table):
    mesh = plsc.VectorSubcoreMesh(core_axis_name="c", subcore_axis_name="s")
    fn = pl.kernel(
        _sc_gather_body,
        mesh=mesh,
        out_type=jax.ShapeDtypeStruct((N, DH), jnp.float32),
        scratch_types=[
            pltpu.VMEM((ROWS_PER_W,), jnp.int32),
            pltpu.VMEM((ROWS_PER_W, DH), jnp.float32),
            pltpu.SemaphoreType.DMA,
        ],
    )
    return fn(idx1d, table)


def _ln(x, g, b, axis, eps=1e-5):
    m = jnp.mean(x, axis=axis, keepdims=True)
    v = jnp.mean((x - m) ** 2, axis=axis, keepdims=True)
    return (x - m) * lax.rsqrt(v + eps) * g + b


def _tc_body(nft_ref, act_ref, res_ref, w1_ref, b1_ref, nng_ref, nnb_ref,
             ln1g_ref, ln1b_ref, wpa_ref, wpb_ref, wpc_ref, bp_ref,
             ln2g_ref, ln2b_ref, out_ref):
    xnt = _ln(nft_ref[...], nng_ref[...], nnb_ref[...], axis=0)
    h = lax.dot_general(w1_ref[...], xnt, (((0,), (0,)), ((), ())),
                        preferred_element_type=jnp.float32)
    h = jnp.maximum(h + b1_ref[...], 0.0)
    num_embt = _ln(h, ln1g_ref[...], ln1b_ref[...], axis=0)
    acc = lax.dot_general(num_embt, wpc_ref[...], (((0,), (0,)), ((), ())),
                          preferred_element_type=jnp.float32)
    acc = acc + jnp.dot(act_ref[...], wpa_ref[...],
                        preferred_element_type=jnp.float32)
    acc = acc + jnp.dot(res_ref[...], wpb_ref[...],
                        preferred_element_type=jnp.float32)
    o = jnp.maximum(acc + bp_ref[...], 0.0)
    out_ref[...] = _ln(o, ln2g_ref[...], ln2b_ref[...], axis=-1)


def _tc_fused(num_feats_t, act_emb, res_emb, nn_g, nn_b, W1, b1, ln1_g,
              ln1_b, Wp, bp, ln2_g, ln2_b, block_n=4096):
    grid = (N // block_n,)
    row_blk = lambda cols: pl.BlockSpec((block_n, cols), lambda i: (i, 0))
    col_blk = lambda rows: pl.BlockSpec((rows, block_n), lambda i: (0, i))
    full = lambda shape: pl.BlockSpec(shape, lambda i: (0, 0))
    return pl.pallas_call(
        _tc_body,
        grid=grid,
        in_specs=[
            col_blk(3),
            row_blk(DH),
            row_blk(DH),
            full((3, D)),
            full((D, 1)),
            full((3, 1)),
            full((3, 1)),
            full((D, 1)),
            full((D, 1)),
            full((DH, D)),
            full((DH, D)),
            full((D, D)),
            full((1, D)),
            full((1, D)),
            full((1, D)),
        ],
        out_specs=row_blk(D),
        out_shape=jax.ShapeDtypeStruct((N, D), jnp.float32),
    )(num_feats_t, act_emb, res_emb, W1, b1.reshape(D, 1),
      nn_g.reshape(3, 1), nn_b.reshape(3, 1), ln1_g.reshape(D, 1),
      ln1_b.reshape(D, 1), Wp[:DH], Wp[DH:D], Wp[D:], bp.reshape(1, D),
      ln2_g.reshape(1, D), ln2_b.reshape(1, D))


def kernel(activities, resources, num_feats, act_table, res_table, nn_g,
           nn_b, W1, b1, ln1_g, ln1_b, Wp, bp, ln2_g, ln2_b):
    act_emb = _sc_gather(activities.astype(jnp.int32), act_table)
    res_emb = _sc_gather(resources.astype(jnp.int32), res_table)
    return _tc_fused(num_feats.T, act_emb, res_emb, nn_g, nn_b, W1, b1,
                     ln1_g, ln1_b, Wp, bp, ln2_g, ln2_b)


# speedup vs baseline: 1.0638x; 1.0003x over previous
import jax
import jax.numpy as jnp
from jax import lax
from jax.experimental import pallas as pl
from jax.experimental.pallas import tpu as pltpu
from jax.experimental.pallas import tpu_sc as plsc

N = 16384
D = 128
DH = 64
NW = 32
ROWS_PER_W = N // NW


def _sc_gather_body(idx, table, out, idx_v, rows, sem):
    wid = lax.axis_index("s") * 2 + lax.axis_index("c")
    base = wid * ROWS_PER_W
    pltpu.sync_copy(idx.at[pl.ds(base, ROWS_PER_W)], idx_v)

    @pl.loop(0, ROWS_PER_W, step=16)
    def _(r):
        v = idx_v[pl.ds(r, 16)]
        for k in range(16):
            pltpu.async_copy(
                table.at[pl.ds(v[k], 1)], rows.at[pl.ds(r + k, 1)], sem)

    pltpu.make_async_copy(table.at[pl.ds(0, ROWS_PER_W)], rows, sem).wait()
    pltpu.sync_copy(rows, out.at[pl.ds(base, ROWS_PER_W)])


def _sc_gather(idx1d, table):
    mesh = plsc.VectorSubcoreMesh(core_axis_name="c", subcore_axis_name="s")
    fn = pl.kernel(
        _sc_gather_body,
        mesh=mesh,
        out_type=jax.ShapeDtypeStruct((N, DH), jnp.float32),
        scratch_types=[
            pltpu.VMEM((ROWS_PER_W,), jnp.int32),
            pltpu.VMEM((ROWS_PER_W, DH), jnp.float32),
            pltpu.SemaphoreType.DMA,
        ],
    )
    return fn(idx1d, table)


def _ln(x, g, b, axis, eps=1e-5):
    m = jnp.mean(x, axis=axis, keepdims=True)
    v = jnp.mean((x - m) ** 2, axis=axis, keepdims=True)
    return (x - m) * lax.rsqrt(v + eps) * g + b


def _tc_body(nft_ref, act_ref, res_ref, w1_ref, b1_ref, nng_ref, nnb_ref,
             ln1g_ref, ln1b_ref, wpa_ref, wpb_ref, wpc_ref, bp_ref,
             ln2g_ref, ln2b_ref, out_ref):
    xnt = _ln(nft_ref[...], nng_ref[...], nnb_ref[...], axis=0)
    h = lax.dot_general(w1_ref[...], xnt, (((0,), (0,)), ((), ())),
                        preferred_element_type=jnp.float32)
    h = jnp.maximum(h + b1_ref[...], 0.0)
    num_embt = _ln(h, ln1g_ref[...], ln1b_ref[...], axis=0)
    acc = lax.dot_general(num_embt, wpc_ref[...], (((0,), (0,)), ((), ())),
                          preferred_element_type=jnp.float32)
    acc = acc + jnp.dot(act_ref[...], wpa_ref[...],
                        preferred_element_type=jnp.float32)
    acc = acc + jnp.dot(res_ref[...], wpb_ref[...],
                        preferred_element_type=jnp.float32)
    o = jnp.maximum(acc + bp_ref[...], 0.0)
    out_ref[...] = _ln(o, ln2g_ref[...], ln2b_ref[...], axis=-1)


def _tc_fused(num_feats_t, act_emb, res_emb, nn_g, nn_b, W1, b1, ln1_g,
              ln1_b, Wp, bp, ln2_g, ln2_b, block_n=4096):
    grid = (N // block_n,)
    row_blk = lambda cols: pl.BlockSpec((block_n, cols), lambda i: (i, 0))
    col_blk = lambda rows: pl.BlockSpec((rows, block_n), lambda i: (0, i))
    full = lambda shape: pl.BlockSpec(shape, lambda i: (0, 0))
    return pl.pallas_call(
        _tc_body,
        grid=grid,
        in_specs=[
            col_blk(3),
            row_blk(DH),
            row_blk(DH),
            full((3, D)),
            full((D, 1)),
            full((3, 1)),
            full((3, 1)),
            full((D, 1)),
            full((D, 1)),
            full((DH, D)),
            full((DH, D)),
            full((D, D)),
            full((1, D)),
            full((1, D)),
            full((1, D)),
        ],
        out_specs=row_blk(D),
        out_shape=jax.ShapeDtypeStruct((N, D), jnp.float32),
    )(num_feats_t, act_emb, res_emb, W1, b1.reshape(D, 1),
      nn_g.reshape(3, 1), nn_b.reshape(3, 1), ln1_g.reshape(D, 1),
      ln1_b.reshape(D, 1), Wp[:DH], Wp[DH:D], Wp[D:], bp.reshape(1, D),
      ln2_g.reshape(1, D), ln2_b.reshape(1, D))


def kernel(activities, resources, num_feats, act_table, res_table, nn_g,
           nn_b, W1, b1, ln1_g, ln1_b, Wp, bp, ln2_g, ln2_b):
    act_emb = _sc_gather(activities.astype(jnp.int32), act_table)
    res_emb = _sc_gather(resources.astype(jnp.int32), res_table)
    return _tc_fused(num_feats.T, act_emb, res_emb, nn_g, nn_b, W1, b1,
                     ln1_g, ln1_b, Wp, bp, ln2_g, ln2_b)
